# baseline (speedup 1.0000x reference)
import functools
import jax
import jax.numpy as jnp
from jax import lax
from jax.experimental import pallas as pl
from jax.experimental.pallas import tpu as pltpu
from jax.experimental.pallas import tpu_sc as plsc

NW = 32
CHUNK = 128
N_PAD = 10240
R_TILE = N_PAD // 16
R_BLK = 1280

_mesh = plsc.VectorSubcoreMesh(core_axis_name="c", subcore_axis_name="s")
_sc_params = pltpu.CompilerParams(needs_layout_passes=False)
_sc_params_lin = pltpu.CompilerParams(needs_layout_passes=False,
                                      use_tc_tiling_on_sc=False)


@functools.lru_cache(maxsize=None)
def _make_deg_kernel(nch):
    @functools.partial(
        pl.kernel,
        out_type=jax.ShapeDtypeStruct((NW, N_PAD), jnp.float32),
        mesh=_mesh,
        compiler_params=_sc_params,
        scratch_types=[
            pltpu.VMEM((nch, CHUNK), jnp.int32),
            pltpu.VMEM((N_PAD,), jnp.float32),
        ],
    )
    def deg_kernel(e_hbm, out_hbm, dst_s, hist_s):
        c = lax.axis_index("c")
        s = lax.axis_index("s")
        wid = c * 16 + s
        pltpu.sync_copy(e_hbm.at[1, pl.ds(wid * nch, nch)], dst_s)
        zeros16 = jnp.zeros((16,), jnp.float32)
        ones16 = jnp.ones((16,), jnp.float32)

        def zbody(i, _):
            hist_s[pl.ds(i * 16, 16)] = zeros16
            return 0

        lax.fori_loop(0, N_PAD // 16, zbody, 0)

        def body(i, _):
            for g in range(CHUNK // 16):
                idx = dst_s[i, pl.ds(g * 16, 16)]
                plsc.addupdate_scatter(hist_s, [idx], ones16)
            return 0

        lax.fori_loop(0, nch, body, 0)
        pltpu.sync_copy(hist_s, out_hbm.at[wid])

    return deg_kernel


NBUF = 2
WIN = 40
NCHT = 80
TOTCH = NW * NCHT


@functools.lru_cache(maxsize=None)
def _make_scatter_kernel(feat):
    @functools.partial(
        pl.kernel,
        out_type=jax.ShapeDtypeStruct((2, N_PAD, feat), jnp.float32),
        mesh=_mesh,
        compiler_params=_sc_params if feat == HID_ else _sc_params_lin,
        scratch_types=[
            pltpu.VMEM((WIN, CHUNK), jnp.int32),
            pltpu.VMEM((WIN, CHUNK), jnp.int32),
            pltpu.VMEM((NBUF, CHUNK, feat), jnp.float32),
            pltpu.VMEM_SHARED((N_PAD, feat), jnp.float32),
            pltpu.SemaphoreType.DMA,
        ],
    )
    def scat_kernel(h_hbm, e_hbm, out_hbm, src_w, dst_w, buf, acc,
                    gsem):
        c = lax.axis_index("c")
        s = lax.axis_index("s")
        wid = c * 16 + s
        start = wid * NCHT

        zeros16 = jnp.zeros((16,), jnp.float32)

        def zbody(r, _):
            for g in range(feat // 16):
                buf[0, r, pl.ds(g * 16, 16)] = zeros16
            return 0

        lax.fori_loop(0, CHUNK, zbody, 0)
        base = s * R_TILE
        for k in range(R_TILE // CHUNK):
            pltpu.sync_copy(buf.at[0], acc.at[pl.ds(base + k * CHUNK, CHUNK)])
        plsc.subcore_barrier()

        def wait_gather():
            pltpu.make_async_copy(h_hbm.at[pl.ds(0, CHUNK)], buf.at[0],
                                  gsem).wait()

        def body(j, _):
            wait_gather()

            @pl.when(j + NBUF - 1 < WIN)
            def _():
                g = j + NBUF - 1
                pltpu.async_copy(h_hbm.at[src_w.at[g]],
                                 buf.at[g % NBUF], gsem)

            pltpu.sync_copy(buf.at[j % NBUF], acc.at[dst_w.at[j]],
                            add=True)
            return 0

        for w in range(NCHT // WIN):
            row = start + w * WIN
            pltpu.sync_copy(e_hbm.at[0, pl.ds(row, WIN)], src_w)
            pltpu.sync_copy(e_hbm.at[1, pl.ds(row, WIN)], dst_w)
            for i in range(NBUF - 1):
                pltpu.async_copy(h_hbm.at[src_w.at[i]], buf.at[i], gsem)
            lax.fori_loop(0, WIN, body, 0)

        plsc.subcore_barrier()
        for k in range(R_TILE // CHUNK):
            pltpu.sync_copy(acc.at[pl.ds(base + k * CHUNK, CHUNK)],
                            out_hbm.at[c, pl.ds(base + k * CHUNK, CHUNK)])

    return scat_kernel


def _dinv_from_hist(hist_blk):
    deg = jnp.sum(hist_blk, axis=0) + 1.0
    return lax.rsqrt(deg)


def _tc_a(hists, x_pad, w1):
    def body(hist_ref, x_ref, w_ref, o_ref):
        dinv = _dinv_from_hist(hist_ref[...])
        h = jnp.dot(x_ref[...], w_ref[...], preferred_element_type=jnp.float32)
        o_ref[...] = h * dinv[:, None]

    return pl.pallas_call(
        body,
        grid=(N_PAD // R_BLK,),
        in_specs=[
            pl.BlockSpec((NW, R_BLK), lambda i: (0, i)),
            pl.BlockSpec((R_BLK, IN_CH_), lambda i: (i, 0)),
            pl.BlockSpec((IN_CH_, HID_), lambda i: (0, 0)),
        ],
        out_specs=pl.BlockSpec((R_BLK, HID_), lambda i: (i, 0)),
        out_shape=jax.ShapeDtypeStruct((N_PAD, HID_), jnp.float32),
    )(hists, x_pad, w1)


def _tc_b(hists, acc, h1, b1, w2):
    def body(hist_ref, acc_ref, h1_ref, b1_ref, w_ref, o_ref):
        dinv = _dinv_from_hist(hist_ref[...])
        t = (acc_ref[0] + acc_ref[1] + h1_ref[...]) * dinv[:, None] + b1_ref[...]
        t = jnp.maximum(t, 0.0)
        h2 = jnp.dot(t, w_ref[...], preferred_element_type=jnp.float32)
        o_ref[...] = h2 * dinv[:, None]

    return pl.pallas_call(
        body,
        grid=(N_PAD // R_BLK,),
        in_specs=[
            pl.BlockSpec((NW, R_BLK), lambda i: (0, i)),
            pl.BlockSpec((2, R_BLK, HID_), lambda i: (0, i, 0)),
            pl.BlockSpec((R_BLK, HID_), lambda i: (i, 0)),
            pl.BlockSpec((1, HID_), lambda i: (0, 0)),
            pl.BlockSpec((HID_, OUT_CH_), lambda i: (0, 0)),
        ],
        out_specs=pl.BlockSpec((R_BLK, OUT_CH_), lambda i: (i, 0)),
        out_shape=jax.ShapeDtypeStruct((N_PAD, OUT_CH_), jnp.float32),
    )(hists, acc, h1, b1, w2)


def _tc_c(hists, acc, h2, b2):
    def body(hist_ref, acc_ref, h2_ref, b2_ref, o_ref):
        dinv = _dinv_from_hist(hist_ref[...])
        o_ref[...] = (acc_ref[0] + acc_ref[1] + h2_ref[...]) * dinv[:, None] + b2_ref[...]

    return pl.pallas_call(
        body,
        grid=(N_PAD // R_BLK,),
        in_specs=[
            pl.BlockSpec((NW, R_BLK), lambda i: (0, i)),
            pl.BlockSpec((2, R_BLK, OUT_CH_), lambda i: (0, i, 0)),
            pl.BlockSpec((R_BLK, OUT_CH_), lambda i: (i, 0)),
            pl.BlockSpec((1, OUT_CH_), lambda i: (0, 0)),
        ],
        out_specs=pl.BlockSpec((R_BLK, OUT_CH_), lambda i: (i, 0)),
        out_shape=jax.ShapeDtypeStruct((N_PAD, OUT_CH_), jnp.float32),
    )(hists, acc, h2, b2)


IN_CH_ = 128
HID_ = 128
OUT_CH_ = 64


def kernel(x, edge_index, W1, b1, W2, b2):
    n, in_ch = x.shape
    e = edge_index.shape[1]
    e_pad = TOTCH * CHUNK

    nreal = e // CHUNK
    pad_iota = n + jnp.arange(e_pad - e, dtype=jnp.int32) % (N_PAD - n)
    pad2 = jnp.broadcast_to(pad_iota.reshape(1, e_pad - e), (2, e_pad - e))
    edges3 = jnp.concatenate(
        [edge_index.astype(jnp.int32), pad2], axis=1).reshape(2, TOTCH, CHUNK)

    b1r = b1.reshape(1, -1)
    out_ch = W2.shape[1]
    b2r = b2.reshape(1, -1)

    hists = _make_deg_kernel(NCHT)(edges3)
    h1 = _tc_a(hists, x, W1)
    acc1 = _make_scatter_kernel(HID_)(h1, edges3)
    h2 = _tc_b(hists, acc1, h1, b1r, W2)
    acc2 = _make_scatter_kernel(OUT_CH_)(h2, edges3)
    out = _tc_c(hists, acc2, h2, b2r)
    return out[:n]


# speedup vs baseline: 1.0115x; 1.0115x over previous
import functools
import jax
import jax.numpy as jnp
from jax import lax
from jax.experimental import pallas as pl
from jax.experimental.pallas import tpu as pltpu
from jax.experimental.pallas import tpu_sc as plsc

NW = 32
CHUNK = 128
N_PAD = 10240
R_TILE = N_PAD // 16
R_BLK = 1280

_mesh = plsc.VectorSubcoreMesh(core_axis_name="c", subcore_axis_name="s")
_sc_params = pltpu.CompilerParams(needs_layout_passes=False)
_sc_params_lin = pltpu.CompilerParams(needs_layout_passes=False,
                                      use_tc_tiling_on_sc=False)


@functools.lru_cache(maxsize=None)
def _make_deg_kernel(nch):
    @functools.partial(
        pl.kernel,
        out_type=jax.ShapeDtypeStruct((NW, N_PAD), jnp.float32),
        mesh=_mesh,
        compiler_params=_sc_params,
        scratch_types=[
            pltpu.VMEM((nch, CHUNK), jnp.int32),
            pltpu.VMEM((N_PAD,), jnp.float32),
        ],
    )
    def deg_kernel(e_hbm, out_hbm, dst_s, hist_s):
        c = lax.axis_index("c")
        s = lax.axis_index("s")
        wid = c * 16 + s
        pltpu.sync_copy(e_hbm.at[1, pl.ds(wid * nch, nch)], dst_s)
        zeros16 = jnp.zeros((16,), jnp.float32)
        ones16 = jnp.ones((16,), jnp.float32)

        def zbody(i, _):
            hist_s[pl.ds(i * 16, 16)] = zeros16
            return 0

        lax.fori_loop(0, N_PAD // 16, zbody, 0)

        def body(i, _):
            for g in range(CHUNK // 16):
                idx = dst_s[i, pl.ds(g * 16, 16)]
                plsc.addupdate_scatter(hist_s, [idx], ones16)
            return 0

        lax.fori_loop(0, nch, body, 0)
        pltpu.sync_copy(hist_s, out_hbm.at[wid])

    return deg_kernel


NBUF = 2
WIN = 40
NCHT = 80
TOTCH = NW * NCHT


@functools.lru_cache(maxsize=None)
def _make_scatter_kernel(feat):
    @functools.partial(
        pl.kernel,
        out_type=jax.ShapeDtypeStruct((2, N_PAD, feat), jnp.float32),
        mesh=_mesh,
        compiler_params=_sc_params if feat == HID_ else _sc_params_lin,
        scratch_types=[
            pltpu.VMEM((WIN, CHUNK), jnp.int32),
            pltpu.VMEM((WIN, CHUNK), jnp.int32),
            pltpu.VMEM((NBUF, CHUNK, feat), jnp.float32),
            pltpu.VMEM_SHARED((N_PAD, feat), jnp.float32),
            pltpu.SemaphoreType.DMA,
        ],
    )
    def scat_kernel(h_hbm, e_hbm, out_hbm, src_w, dst_w, buf, acc,
                    gsem):
        c = lax.axis_index("c")
        s = lax.axis_index("s")
        wid = c * 16 + s
        start = wid * NCHT

        zeros16 = jnp.zeros((16,), jnp.float32)

        def zbody(r, _):
            for g in range(feat // 16):
                buf[0, r, pl.ds(g * 16, 16)] = zeros16
            return 0

        lax.fori_loop(0, CHUNK, zbody, 0)
        base = s * R_TILE
        for k in range(R_TILE // CHUNK):
            pltpu.sync_copy(buf.at[0], acc.at[pl.ds(base + k * CHUNK, CHUNK)])
        plsc.subcore_barrier()

        def wait_gather():
            pltpu.make_async_copy(h_hbm.at[pl.ds(0, CHUNK)], buf.at[0],
                                  gsem).wait()

        def body(j, _):
            wait_gather()

            @pl.when(j + NBUF - 1 < WIN)
            def _():
                g = j + NBUF - 1
                pltpu.async_copy(h_hbm.at[src_w.at[g]],
                                 buf.at[g % NBUF], gsem)

            pltpu.sync_copy(buf.at[j % NBUF], acc.at[dst_w.at[j]],
                            add=True)
            return 0

        for w in range(NCHT // WIN):
            row = start + w * WIN
            pltpu.sync_copy(e_hbm.at[0, pl.ds(row, WIN)], src_w)
            pltpu.sync_copy(e_hbm.at[1, pl.ds(row, WIN)], dst_w)
            for i in range(NBUF - 1):
                pltpu.async_copy(h_hbm.at[src_w.at[i]], buf.at[i], gsem)
            lax.fori_loop(0, WIN, body, 0)

        plsc.subcore_barrier()
        for k in range(R_TILE // CHUNK):
            pltpu.sync_copy(acc.at[pl.ds(base + k * CHUNK, CHUNK)],
                            out_hbm.at[c, pl.ds(base + k * CHUNK, CHUNK)])

    return scat_kernel


def _dinv_from_hist(hist_blk):
    deg = jnp.sum(hist_blk, axis=0) + 1.0
    return lax.rsqrt(deg)


def _tc_a(hists, x_pad, w1):
    def body(hist_ref, x_ref, w_ref, o_ref):
        dinv = _dinv_from_hist(hist_ref[...])
        h = jnp.dot(x_ref[...], w_ref[...], preferred_element_type=jnp.float32)
        o_ref[...] = h * dinv[:, None]

    return pl.pallas_call(
        body,
        grid=(N_PAD // R_BLK,),
        in_specs=[
            pl.BlockSpec((NW, R_BLK), lambda i: (0, i)),
            pl.BlockSpec((R_BLK, IN_CH_), lambda i: (i, 0)),
            pl.BlockSpec((IN_CH_, HID_), lambda i: (0, 0)),
        ],
        out_specs=pl.BlockSpec((R_BLK, HID_), lambda i: (i, 0)),
        out_shape=jax.ShapeDtypeStruct((N_PAD, HID_), jnp.float32),
    )(hists, x_pad, w1)


def _tc_b(hists, acc, h1, b1, w2):
    def body(hist_ref, acc_ref, h1_ref, b1_ref, w_ref, o_ref):
        dinv = _dinv_from_hist(hist_ref[...])
        t = (acc_ref[0] + acc_ref[1] + h1_ref[...]) * dinv[:, None] + b1_ref[...]
        t = jnp.maximum(t, 0.0)
        h2 = jnp.dot(t, w_ref[...], preferred_element_type=jnp.float32)
        o_ref[...] = h2 * dinv[:, None]

    return pl.pallas_call(
        body,
        grid=(N_PAD // R_BLK,),
        in_specs=[
            pl.BlockSpec((NW, R_BLK), lambda i: (0, i)),
            pl.BlockSpec((2, R_BLK, HID_), lambda i: (0, i, 0)),
            pl.BlockSpec((R_BLK, HID_), lambda i: (i, 0)),
            pl.BlockSpec((1, HID_), lambda i: (0, 0)),
            pl.BlockSpec((HID_, OUT_CH_), lambda i: (0, 0)),
        ],
        out_specs=pl.BlockSpec((R_BLK, OUT_CH_), lambda i: (i, 0)),
        out_shape=jax.ShapeDtypeStruct((N_PAD, OUT_CH_), jnp.float32),
    )(hists, acc, h1, b1, w2)


def _tc_c(hists, acc, h2, b2):
    def body(hist_ref, acc_ref, h2_ref, b2_ref, o_ref):
        dinv = _dinv_from_hist(hist_ref[...])
        o_ref[...] = (acc_ref[0] + acc_ref[1] + h2_ref[...]) * dinv[:, None] + b2_ref[...]

    return pl.pallas_call(
        body,
        grid=(N_PAD // R_BLK,),
        in_specs=[
            pl.BlockSpec((NW, R_BLK), lambda i: (0, i)),
            pl.BlockSpec((2, R_BLK, OUT_CH_), lambda i: (0, i, 0)),
            pl.BlockSpec((R_BLK, OUT_CH_), lambda i: (i, 0)),
            pl.BlockSpec((1, OUT_CH_), lambda i: (0, 0)),
        ],
        out_specs=pl.BlockSpec((R_BLK, OUT_CH_), lambda i: (i, 0)),
        out_shape=jax.ShapeDtypeStruct((N_PAD, OUT_CH_), jnp.float32),
    )(hists, acc, h2, b2)


IN_CH_ = 128
HID_ = 128
OUT_CH_ = 64


def kernel(x, edge_index, W1, b1, W2, b2):
    n, in_ch = x.shape
    e = edge_index.shape[1]
    e_pad = TOTCH * CHUNK

    nreal = e // CHUNK
    pad_iota = n + jnp.arange(e_pad - e, dtype=jnp.int32) % (N_PAD - n)
    pad3 = jnp.broadcast_to(pad_iota.reshape(1, TOTCH - nreal, CHUNK),
                            (2, TOTCH - nreal, CHUNK))
    edges3 = jnp.concatenate(
        [edge_index.astype(jnp.int32).reshape(2, nreal, CHUNK), pad3], axis=1)

    b1r = b1.reshape(1, -1)
    out_ch = W2.shape[1]
    b2r = b2.reshape(1, -1)

    hists = _make_deg_kernel(NCHT)(edges3)
    h1 = _tc_a(hists, x, W1)
    acc1 = _make_scatter_kernel(HID_)(h1, edges3)
    h2 = _tc_b(hists, acc1, h1, b1r, W2)
    acc2 = _make_scatter_kernel(OUT_CH_)(h2, edges3)
    out = _tc_c(hists, acc2, h2, b2r)
    return out[:n]
